# baseline (speedup 1.0000x reference)
import functools

import jax
import jax.numpy as jnp
import numpy as np
from jax import lax
from jax.experimental import pallas as pl
from jax.experimental.pallas import tpu as pltpu
from jax.experimental.pallas import tpu_sc as plsc

_PROJ_H = 12
_PROJ_W = 12
_N_ACTIONS = 96
_ORIGINS = {0: (0, 4), 1: (4, 2), 2: (4, 6), 3: (8, 0), 4: (8, 4), 5: (8, 8)}


def _build_proj_index():
    idx = []
    for g in range(_N_ACTIONS):
        b = g // 16
        loc = g % 16
        r, c = (loc // 4, loc % 4)
        br, bc = _ORIGINS[b]
        idx.append((br + r) * _PROJ_W + (bc + c))
    return np.asarray(idx, dtype=np.int32)

_PROJ_IDX_NP = _build_proj_index()
_VALID_NP = np.zeros(_PROJ_H * _PROJ_W, dtype=np.float32)
_VALID_NP[_PROJ_IDX_NP] = 1.0

_NCELL = _PROJ_H * _PROJ_W
_OUTW = 3 * _NCELL
_NC = 2
_NS = 16
_NW = _NC * _NS
_CHUNK = 64


def _sc_body(obs_hbm, idx_hbm, valid_hbm, out_hbm, idx_v, valid_v, in_v, out_v,
             *, nchunks):
    wid = lax.axis_index("s") * _NC + lax.axis_index("c")
    base = wid * (nchunks * _CHUNK)

    pltpu.sync_copy(idx_hbm, idx_v)
    pltpu.sync_copy(valid_hbm, valid_v)

    pos0 = [idx_v[pl.ds(16 * j, 16)] for j in range(6)]
    pos1 = [p + _NCELL for p in pos0]
    valid_regs = [valid_v[pl.ds(16 * k, 16)] for k in range(9)]
    zero = jnp.zeros((16,), jnp.float32)
    one = jnp.ones((16,), jnp.float32)

    def prefill(r, carry):
        rb = r * _OUTW
        for k in range(9):
            out_v[pl.ds(rb + 16 * k, 16)] = zero
            out_v[pl.ds(rb + _NCELL + 16 * k, 16)] = zero
            out_v[pl.ds(rb + 2 * _NCELL + 16 * k, 16)] = valid_regs[k]
        return carry

    lax.fori_loop(0, _CHUNK, prefill, 0)

    def chunk_body(ci, carry):
        rbase = base + ci * _CHUNK
        pltpu.sync_copy(obs_hbm.at[pl.ds(rbase * _N_ACTIONS, _CHUNK * _N_ACTIONS)],
                        in_v)

        def row_body(r, inner):
            rvec = jnp.full((16,), r * _OUTW, jnp.int32)
            for j in range(6):
                x = in_v[pl.ds(r * _N_ACTIONS + 16 * j, 16)]
                a = jnp.where(x > 0.5, one, zero)
                o = jnp.where(x < -0.5, one, zero)
                plsc.store_scatter(out_v, [rvec + pos0[j]], a)
                plsc.store_scatter(out_v, [rvec + pos1[j]], o)
            return inner

        lax.fori_loop(0, _CHUNK, row_body, 0)
        pltpu.sync_copy(out_v, out_hbm.at[pl.ds(rbase * _OUTW, _CHUNK * _OUTW)])
        return carry

    lax.fori_loop(0, nchunks, chunk_body, 0)


@jax.jit
def kernel(observation):
    if observation.ndim == 1:
        observation = observation[None, :]
    bsz = observation.shape[0]

    step = _NW * _CHUNK
    bpad = ((bsz + step - 1) // step) * step
    obs = observation.astype(jnp.float32)
    if bpad != bsz:
        obs = jnp.pad(obs, ((0, bpad - bsz), (0, 0)))
    nchunks = bpad // step

    idx = jnp.asarray(_PROJ_IDX_NP)
    valid = jnp.asarray(_VALID_NP)

    run = pl.kernel(
        functools.partial(_sc_body, nchunks=nchunks),
        out_type=jax.ShapeDtypeStruct((bpad * _OUTW,), jnp.float32),
        mesh=plsc.VectorSubcoreMesh(core_axis_name="c", subcore_axis_name="s"),
        compiler_params=pltpu.CompilerParams(
            needs_layout_passes=False, use_tc_tiling_on_sc=False),
        scratch_types=[
            pltpu.VMEM((_N_ACTIONS,), jnp.int32),
            pltpu.VMEM((_NCELL,), jnp.float32),
            pltpu.VMEM((_CHUNK * _N_ACTIONS,), jnp.float32),
            pltpu.VMEM((_CHUNK * _OUTW,), jnp.float32),
        ],
    )
    flat = run(obs.reshape(bpad * _N_ACTIONS), idx, valid)
    board = flat.reshape(bpad, _OUTW)[:bsz].reshape(bsz, 3, _PROJ_H, _PROJ_W)
    return board.astype(observation.dtype)


# speedup vs baseline: 1.0147x; 1.0147x over previous
import functools

import jax
import jax.numpy as jnp
import numpy as np
from jax import lax
from jax.experimental import pallas as pl
from jax.experimental.pallas import tpu as pltpu
from jax.experimental.pallas import tpu_sc as plsc

_PROJ_H = 12
_PROJ_W = 12
_N_ACTIONS = 96
_ORIGINS = {0: (0, 4), 1: (4, 2), 2: (4, 6), 3: (8, 0), 4: (8, 4), 5: (8, 8)}


def _build_proj_index():
    idx = []
    for g in range(_N_ACTIONS):
        b = g // 16
        loc = g % 16
        r, c = (loc // 4, loc % 4)
        br, bc = _ORIGINS[b]
        idx.append((br + r) * _PROJ_W + (bc + c))
    return np.asarray(idx, dtype=np.int32)

_PROJ_IDX_NP = _build_proj_index()
_VALID_NP = np.zeros(_PROJ_H * _PROJ_W, dtype=np.float32)
_VALID_NP[_PROJ_IDX_NP] = 1.0

_NCELL = _PROJ_H * _PROJ_W
_OUTW = 3 * _NCELL
_NC = 2
_NS = 16
_NW = _NC * _NS
_CHUNK = 64
_U = 4


def _sc_body(obs_hbm, idx_hbm, valid_hbm, out_hbm,
             idx_v, valid_v, in_v0, in_v1, out_v0, out_v1,
             semi0, semi1, semo0, semo1, *, nchunks):
    wid = lax.axis_index("s") * _NC + lax.axis_index("c")
    base = wid * (nchunks * _CHUNK)

    pltpu.sync_copy(idx_hbm, idx_v)
    pltpu.sync_copy(valid_hbm, valid_v)

    pos0 = [idx_v[pl.ds(16 * j, 16)] for j in range(6)]
    pos1 = [p + _NCELL for p in pos0]
    valid_regs = [valid_v[pl.ds(16 * k, 16)] for k in range(9)]
    zero = jnp.zeros((16,), jnp.float32)
    one = jnp.ones((16,), jnp.float32)

    ins = [in_v0, in_v1]
    outs = [out_v0, out_v1]
    semi = [semi0, semi1]
    semo = [semo0, semo1]

    def prefill(r, carry):
        rb = r * _OUTW
        for buf in outs:
            for k in range(9):
                buf[pl.ds(rb + 16 * k, 16)] = zero
                buf[pl.ds(rb + _NCELL + 16 * k, 16)] = zero
                buf[pl.ds(rb + 2 * _NCELL + 16 * k, 16)] = valid_regs[k]
        return carry

    lax.fori_loop(0, _CHUNK, prefill, 0)

    def in_slice(ci):
        return obs_hbm.at[pl.ds((base + ci * _CHUNK) * _N_ACTIONS,
                                _CHUNK * _N_ACTIONS)]

    def out_slice(ci):
        return out_hbm.at[pl.ds((base + ci * _CHUNK) * _OUTW,
                                _CHUNK * _OUTW)]

    in_d = [None, None]
    out_d = [None, None]
    in_d[0] = pltpu.async_copy(in_slice(0), ins[0], semi[0])

    def make_compute(in_buf, out_buf):
        def body(i, carry):
            rvec, roff = carry
            for u in range(_U):
                rv = rvec + u * _OUTW
                ro = roff + u * _N_ACTIONS
                for j in range(6):
                    x = in_buf[pl.ds(ro + 16 * j, 16)]
                    a = jnp.where(x > 0.5, one, zero)
                    o = jnp.where(x < -0.5, one, zero)
                    plsc.store_scatter(out_buf, [rv + pos0[j]], a)
                    plsc.store_scatter(out_buf, [rv + pos1[j]], o)
            return rvec + _U * _OUTW, roff + _U * _N_ACTIONS
        return body

    rvec0 = jnp.zeros((16,), jnp.int32)
    for ci in range(nchunks):
        p = ci & 1
        q = (ci + 1) & 1
        if ci + 1 < nchunks:
            in_d[q] = pltpu.async_copy(in_slice(ci + 1), ins[q], semi[q])
        in_d[p].wait()
        if out_d[p] is not None:
            out_d[p].wait()
        lax.fori_loop(0, _CHUNK // _U, make_compute(ins[p], outs[p]),
                      (rvec0, 0))
        out_d[p] = pltpu.async_copy(outs[p], out_slice(ci), semo[p])

    for d in out_d:
        if d is not None:
            d.wait()


@jax.jit
def kernel(observation):
    if observation.ndim == 1:
        observation = observation[None, :]
    bsz = observation.shape[0]

    step = _NW * _CHUNK
    bpad = ((bsz + step - 1) // step) * step
    obs = observation.astype(jnp.float32)
    if bpad != bsz:
        obs = jnp.pad(obs, ((0, bpad - bsz), (0, 0)))
    nchunks = bpad // step

    idx = jnp.asarray(_PROJ_IDX_NP)
    valid = jnp.asarray(_VALID_NP)

    run = pl.kernel(
        functools.partial(_sc_body, nchunks=nchunks),
        out_type=jax.ShapeDtypeStruct((bpad * _OUTW,), jnp.float32),
        mesh=plsc.VectorSubcoreMesh(core_axis_name="c", subcore_axis_name="s"),
        compiler_params=pltpu.CompilerParams(
            needs_layout_passes=False, use_tc_tiling_on_sc=False),
        scratch_types=[
            pltpu.VMEM((_N_ACTIONS,), jnp.int32),
            pltpu.VMEM((_NCELL,), jnp.float32),
            pltpu.VMEM((_CHUNK * _N_ACTIONS,), jnp.float32),
            pltpu.VMEM((_CHUNK * _N_ACTIONS,), jnp.float32),
            pltpu.VMEM((_CHUNK * _OUTW,), jnp.float32),
            pltpu.VMEM((_CHUNK * _OUTW,), jnp.float32),
            pltpu.SemaphoreType.DMA,
            pltpu.SemaphoreType.DMA,
            pltpu.SemaphoreType.DMA,
            pltpu.SemaphoreType.DMA,
        ],
    )
    flat = run(obs.reshape(bpad * _N_ACTIONS), idx, valid)
    board = flat.reshape(bpad, _OUTW)[:bsz].reshape(bsz, 3, _PROJ_H, _PROJ_W)
    return board.astype(observation.dtype)


# speedup vs baseline: 8.8877x; 8.7587x over previous
import functools

import jax
import jax.numpy as jnp
import numpy as np
from jax import lax
from jax.experimental import pallas as pl
from jax.experimental.pallas import tpu as pltpu
from jax.experimental.pallas import tpu_sc as plsc

_PROJ_H = 12
_PROJ_W = 12
_N_ACTIONS = 96
_NCELL = _PROJ_H * _PROJ_W
_ORIGINS = {0: (0, 4), 1: (4, 2), 2: (4, 6), 3: (8, 0), 4: (8, 4), 5: (8, 8)}


def _build_proj_index():
    idx = []
    for g in range(_N_ACTIONS):
        b = g // 16
        loc = g % 16
        r, c = (loc // 4, loc % 4)
        br, bc = _ORIGINS[b]
        idx.append((br + r) * _PROJ_W + (bc + c))
    return np.asarray(idx, dtype=np.int32)

_PROJ_IDX_NP = _build_proj_index()
_SRC = [None] * _NCELL
for _g, _p in enumerate(_PROJ_IDX_NP):
    _SRC[int(_p)] = _g
_VALID_CELLS = [p for p in range(_NCELL) if _SRC[p] is not None]

_NC = 2
_NS = 16
_NW = _NC * _NS
_BCHUNK = 128
_LGRP = _BCHUNK // 16
_NSLAB = 3 * _PROJ_H


def _sc_body(obs_hbm, out_hbm, in_v0, in_v1, out_v, semi0, semi1, semo,
             *, nchunks):
    wid = lax.axis_index("s") * _NC + lax.axis_index("c")
    base = wid * (nchunks * _BCHUNK)

    zero = jnp.zeros((16,), jnp.float32)
    one = jnp.ones((16,), jnp.float32)
    ins = [in_v0, in_v1]
    semi = [semi0, semi1]

    def prefill(l, carry):
        col = l * 16
        for p in range(_NCELL):
            if _SRC[p] is None:
                out_v[p, pl.ds(col, 16)] = zero
                out_v[_NCELL + p, pl.ds(col, 16)] = zero
                out_v[2 * _NCELL + p, pl.ds(col, 16)] = zero
            else:
                out_v[2 * _NCELL + p, pl.ds(col, 16)] = one
        return carry

    lax.fori_loop(0, _LGRP, prefill, 0)

    def in_slice(ci):
        return obs_hbm.at[:, pl.ds(base + ci * _BCHUNK, _BCHUNK)]

    in_d = [None, None]
    out_d = []
    in_d[0] = pltpu.async_copy(in_slice(0), ins[0], semi[0])

    def make_compute(in_buf):
        def body(l, carry):
            col = l * 16
            for p in _VALID_CELLS:
                g = _SRC[p]
                x = in_buf[g, pl.ds(col, 16)]
                out_v[p, pl.ds(col, 16)] = jnp.where(x > 0.5, one, zero)
                out_v[_NCELL + p, pl.ds(col, 16)] = jnp.where(
                    x < -0.5, one, zero)
            return carry
        return body

    for ci in range(nchunks):
        p = ci & 1
        q = (ci + 1) & 1
        if ci + 1 < nchunks:
            in_d[q] = pltpu.async_copy(in_slice(ci + 1), ins[q], semi[q])
        in_d[p].wait()
        for d in out_d:
            d.wait()
        lax.fori_loop(0, _LGRP, make_compute(ins[p]), 0)
        bcol = base + ci * _BCHUNK
        out_d = [
            pltpu.async_copy(out_v.at[pl.ds(s * _PROJ_W, _PROJ_W)],
                             out_hbm.at[s, :, pl.ds(bcol, _BCHUNK)], semo)
            for s in range(_NSLAB)
        ]

    for d in out_d:
        d.wait()


@jax.jit
def kernel(observation):
    if observation.ndim == 1:
        observation = observation[None, :]
    bsz = observation.shape[0]

    step = _NW * _BCHUNK
    bpad = ((bsz + step - 1) // step) * step
    obs_t = observation.astype(jnp.float32).T
    if bpad != bsz:
        obs_t = jnp.pad(obs_t, ((0, 0), (0, bpad - bsz)))
    nchunks = bpad // step

    run = pl.kernel(
        functools.partial(_sc_body, nchunks=nchunks),
        out_type=jax.ShapeDtypeStruct((_NSLAB, _PROJ_W, bpad), jnp.float32),
        mesh=plsc.VectorSubcoreMesh(core_axis_name="c", subcore_axis_name="s"),
        compiler_params=pltpu.CompilerParams(
            needs_layout_passes=False, use_tc_tiling_on_sc=True),
        scratch_types=[
            pltpu.VMEM((_N_ACTIONS, _BCHUNK), jnp.float32),
            pltpu.VMEM((_N_ACTIONS, _BCHUNK), jnp.float32),
            pltpu.VMEM((3 * _NCELL, _BCHUNK), jnp.float32),
            pltpu.SemaphoreType.DMA,
            pltpu.SemaphoreType.DMA,
            pltpu.SemaphoreType.DMA,
        ],
    )
    out3 = run(obs_t)
    board = jnp.transpose(out3.reshape(3, _PROJ_H, _PROJ_W, bpad),
                          (3, 0, 1, 2))[:bsz]
    return board.astype(observation.dtype)


# speedup vs baseline: 8.9382x; 1.0057x over previous
import functools

import jax
import jax.numpy as jnp
import numpy as np
from jax import lax
from jax.experimental import pallas as pl
from jax.experimental.pallas import tpu as pltpu
from jax.experimental.pallas import tpu_sc as plsc

_PROJ_H = 12
_PROJ_W = 12
_N_ACTIONS = 96
_NCELL = _PROJ_H * _PROJ_W
_ORIGINS = {0: (0, 4), 1: (4, 2), 2: (4, 6), 3: (8, 0), 4: (8, 4), 5: (8, 8)}


def _build_proj_index():
    idx = []
    for g in range(_N_ACTIONS):
        b = g // 16
        loc = g % 16
        r, c = (loc // 4, loc % 4)
        br, bc = _ORIGINS[b]
        idx.append((br + r) * _PROJ_W + (bc + c))
    return np.asarray(idx, dtype=np.int32)

_PROJ_IDX_NP = _build_proj_index()
_SRC = [None] * _NCELL
for _g, _p in enumerate(_PROJ_IDX_NP):
    _SRC[int(_p)] = _g
_VALID_CELLS = [p for p in range(_NCELL) if _SRC[p] is not None]

_NC = 2
_NS = 16
_NW = _NC * _NS
_BCHUNK = 128
_LGRP = _BCHUNK // 16
_NSLAB = 3 * _PROJ_H


def _sc_body(obs_hbm, out_hbm, in_v0, in_v1, out_v0, out_v1, p2_v,
             semi0, semi1, semo0, semo1, semp2, *, nchunks):
    wid = lax.axis_index("s") * _NC + lax.axis_index("c")
    base = wid * (nchunks * _BCHUNK)

    zero = jnp.zeros((16,), jnp.float32)
    one = jnp.ones((16,), jnp.float32)
    ins = [in_v0, in_v1]
    outs = [out_v0, out_v1]
    semi = [semi0, semi1]
    semo = [semo0, semo1]

    def in_slice(ci):
        return obs_hbm.at[:, pl.ds(base + ci * _BCHUNK, _BCHUNK)]

    in_d = [None, None]
    in_d[0] = pltpu.async_copy(in_slice(0), ins[0], semi[0])

    def prefill(l, carry):
        col = l * 16
        for p in range(_NCELL):
            if _SRC[p] is None:
                for buf in outs:
                    buf[p, pl.ds(col, 16)] = zero
                    buf[_NCELL + p, pl.ds(col, 16)] = zero
                p2_v[p, pl.ds(col, 16)] = zero
            else:
                p2_v[p, pl.ds(col, 16)] = one
        return carry

    lax.fori_loop(0, _LGRP, prefill, 0)

    def make_compute(in_buf, out_buf):
        def body(l, carry):
            col = l * 16
            for p in _VALID_CELLS:
                g = _SRC[p]
                x = in_buf[g, pl.ds(col, 16)]
                out_buf[p, pl.ds(col, 16)] = jnp.where(x > 0.5, one, zero)
                out_buf[_NCELL + p, pl.ds(col, 16)] = jnp.where(
                    x < -0.5, one, zero)
            return carry
        return body

    out_d = [[], []]
    p2_d = []
    for ci in range(nchunks):
        pb = ci & 1
        qb = (ci + 1) & 1
        if ci + 1 < nchunks:
            in_d[qb] = pltpu.async_copy(in_slice(ci + 1), ins[qb], semi[qb])
        in_d[pb].wait()
        for d in out_d[pb]:
            d.wait()
        for d in p2_d:
            d.wait()
        p2_d = []
        lax.fori_loop(0, _LGRP, make_compute(ins[pb], outs[pb]), 0)
        bcol = base + ci * _BCHUNK
        out_d[pb] = [
            pltpu.async_copy(outs[pb].at[pl.ds(p0 * _PROJ_W, _PROJ_W)],
                             out_hbm.at[s, :, pl.ds(bcol, _BCHUNK)], semo[pb])
            for s, p0 in [(h, h) for h in range(_PROJ_H)]
            + [(12 + h, 12 + h) for h in range(_PROJ_H)]
        ]
        p2_d = [
            pltpu.async_copy(p2_v.at[pl.ds(h * _PROJ_W, _PROJ_W)],
                             out_hbm.at[24 + h, :, pl.ds(bcol, _BCHUNK)],
                             semp2)
            for h in range(_PROJ_H)
        ]

    for ds in out_d:
        for d in ds:
            d.wait()
    for d in p2_d:
        d.wait()


@jax.jit
def kernel(observation):
    if observation.ndim == 1:
        observation = observation[None, :]
    bsz = observation.shape[0]

    step = _NW * _BCHUNK
    bpad = ((bsz + step - 1) // step) * step
    obs_t = observation.astype(jnp.float32).T
    if bpad != bsz:
        obs_t = jnp.pad(obs_t, ((0, 0), (0, bpad - bsz)))
    nchunks = bpad // step

    run = pl.kernel(
        functools.partial(_sc_body, nchunks=nchunks),
        out_type=jax.ShapeDtypeStruct((_NSLAB, _PROJ_W, bpad), jnp.float32),
        mesh=plsc.VectorSubcoreMesh(core_axis_name="c", subcore_axis_name="s"),
        compiler_params=pltpu.CompilerParams(
            needs_layout_passes=False, use_tc_tiling_on_sc=True,
            skip_device_barrier=True, disable_bounds_checks=True,
            disable_semaphore_checks=True),
        scratch_types=[
            pltpu.VMEM((_N_ACTIONS, _BCHUNK), jnp.float32),
            pltpu.VMEM((_N_ACTIONS, _BCHUNK), jnp.float32),
            pltpu.VMEM((2 * _NCELL, _BCHUNK), jnp.float32),
            pltpu.VMEM((2 * _NCELL, _BCHUNK), jnp.float32),
            pltpu.VMEM((_NCELL, _BCHUNK), jnp.float32),
            pltpu.SemaphoreType.DMA,
            pltpu.SemaphoreType.DMA,
            pltpu.SemaphoreType.DMA,
            pltpu.SemaphoreType.DMA,
            pltpu.SemaphoreType.DMA,
        ],
    )
    out3 = run(obs_t)
    board = jnp.transpose(out3.reshape(3, _PROJ_H, _PROJ_W, bpad),
                          (3, 0, 1, 2))[:bsz]
    return board.astype(observation.dtype)


# speedup vs baseline: 9.0131x; 1.0084x over previous
import functools

import jax
import jax.numpy as jnp
import numpy as np
from jax import lax
from jax.experimental import pallas as pl
from jax.experimental.pallas import tpu as pltpu
from jax.experimental.pallas import tpu_sc as plsc

_PROJ_H = 12
_PROJ_W = 12
_N_ACTIONS = 96
_NCELL = _PROJ_H * _PROJ_W
_ORIGINS = {0: (0, 4), 1: (4, 2), 2: (4, 6), 3: (8, 0), 4: (8, 4), 5: (8, 8)}


def _build_proj_index():
    idx = []
    for g in range(_N_ACTIONS):
        b = g // 16
        loc = g % 16
        r, c = (loc // 4, loc % 4)
        br, bc = _ORIGINS[b]
        idx.append((br + r) * _PROJ_W + (bc + c))
    return np.asarray(idx, dtype=np.int32)

_PROJ_IDX_NP = _build_proj_index()
_SRC = [None] * _NCELL
for _g, _p in enumerate(_PROJ_IDX_NP):
    _SRC[int(_p)] = _g
_VALID_CELLS = [p for p in range(_NCELL) if _SRC[p] is not None]

_NC = 2
_NS = 16
_NW = _NC * _NS
_BCHUNK = 128
_LGRP = _BCHUNK // 16
_NSLAB = 3 * _PROJ_H


def _sc_body(obs_hbm, out_hbm, in_v0, in_v1, out_v0, out_v1, p2_v,
             semi0, semi1, semo0, semo1, semp2, *, nchunks):
    wid = lax.axis_index("s") * _NC + lax.axis_index("c")
    base = wid * (nchunks * _BCHUNK)

    zero = jnp.zeros((16,), jnp.float32)
    one = jnp.ones((16,), jnp.float32)
    ins = [in_v0, in_v1]
    outs = [out_v0, out_v1]
    semi = [semi0, semi1]
    semo = [semo0, semo1]

    def in_slice(ci):
        return obs_hbm.at[:, pl.ds(base + ci * _BCHUNK, _BCHUNK)]

    in_d = [None, None]
    in_d[0] = pltpu.async_copy(in_slice(0), ins[0], semi[0])

    @plsc.parallel_loop(0, _LGRP)
    def _(l):
        col = l * 16
        for p in range(_NCELL):
            if _SRC[p] is None:
                for buf in outs:
                    buf[p, pl.ds(col, 16)] = zero
                    buf[_NCELL + p, pl.ds(col, 16)] = zero
                p2_v[p, pl.ds(col, 16)] = zero
            else:
                p2_v[p, pl.ds(col, 16)] = one

    def run_compute(in_buf, out_buf):
        @plsc.parallel_loop(0, _LGRP)
        def _(l):
            col = l * 16
            for p in _VALID_CELLS:
                g = _SRC[p]
                x = in_buf[g, pl.ds(col, 16)]
                out_buf[p, pl.ds(col, 16)] = jnp.where(x > 0.5, one, zero)
                out_buf[_NCELL + p, pl.ds(col, 16)] = jnp.where(
                    x < -0.5, one, zero)

    out_d = [[], []]
    p2_d = []
    for ci in range(nchunks):
        pb = ci & 1
        qb = (ci + 1) & 1
        if ci + 1 < nchunks:
            in_d[qb] = pltpu.async_copy(in_slice(ci + 1), ins[qb], semi[qb])
        in_d[pb].wait()
        for d in out_d[pb]:
            d.wait()
        for d in p2_d:
            d.wait()
        p2_d = []
        run_compute(ins[pb], outs[pb])
        bcol = base + ci * _BCHUNK
        out_d[pb] = [
            pltpu.async_copy(outs[pb].at[pl.ds(p0 * _PROJ_W, _PROJ_W)],
                             out_hbm.at[s, :, pl.ds(bcol, _BCHUNK)], semo[pb])
            for s, p0 in [(h, h) for h in range(_PROJ_H)]
            + [(12 + h, 12 + h) for h in range(_PROJ_H)]
        ]
        p2_d = [
            pltpu.async_copy(p2_v.at[pl.ds(h * _PROJ_W, _PROJ_W)],
                             out_hbm.at[24 + h, :, pl.ds(bcol, _BCHUNK)],
                             semp2)
            for h in range(_PROJ_H)
        ]

    for ds in out_d:
        for d in ds:
            d.wait()
    for d in p2_d:
        d.wait()


@jax.jit
def kernel(observation):
    if observation.ndim == 1:
        observation = observation[None, :]
    bsz = observation.shape[0]

    step = _NW * _BCHUNK
    bpad = ((bsz + step - 1) // step) * step
    obs_t = observation.astype(jnp.float32).T
    if bpad != bsz:
        obs_t = jnp.pad(obs_t, ((0, 0), (0, bpad - bsz)))
    nchunks = bpad // step

    run = pl.kernel(
        functools.partial(_sc_body, nchunks=nchunks),
        out_type=jax.ShapeDtypeStruct((_NSLAB, _PROJ_W, bpad), jnp.float32),
        mesh=plsc.VectorSubcoreMesh(core_axis_name="c", subcore_axis_name="s"),
        compiler_params=pltpu.CompilerParams(
            needs_layout_passes=False, use_tc_tiling_on_sc=True,
            skip_device_barrier=True, disable_bounds_checks=True,
            disable_semaphore_checks=True),
        scratch_types=[
            pltpu.VMEM((_N_ACTIONS, _BCHUNK), jnp.float32),
            pltpu.VMEM((_N_ACTIONS, _BCHUNK), jnp.float32),
            pltpu.VMEM((2 * _NCELL, _BCHUNK), jnp.float32),
            pltpu.VMEM((2 * _NCELL, _BCHUNK), jnp.float32),
            pltpu.VMEM((_NCELL, _BCHUNK), jnp.float32),
            pltpu.SemaphoreType.DMA,
            pltpu.SemaphoreType.DMA,
            pltpu.SemaphoreType.DMA,
            pltpu.SemaphoreType.DMA,
            pltpu.SemaphoreType.DMA,
        ],
    )
    out3 = run(obs_t)
    board = jnp.transpose(out3.reshape(3, _PROJ_H, _PROJ_W, bpad),
                          (3, 0, 1, 2))[:bsz]
    return board.astype(observation.dtype)
